# baseline (speedup 1.0000x reference)
import functools

import jax
import jax.numpy as jnp
from jax import lax
from jax.experimental import pallas as pl
from jax.experimental.pallas import tpu as pltpu
from jax.experimental.pallas import tpu_sc as plsc

D = 512
SIZE = 65536
NC = 2
NS = 16
NW = NC * NS
ROWS_PER_W = SIZE // NW
CHUNK = 64
NBUF = 2
NCHUNK = ROWS_PER_W // CHUNK
NGROUPS = NCHUNK // NBUF
NSUB = CHUNK // 16
DCH = D // 16

_mesh = plsc.VectorSubcoreMesh(core_axis_name="c", subcore_axis_name="s")


@functools.partial(
    pl.kernel,
    mesh=_mesh,
    compiler_params=pltpu.CompilerParams(needs_layout_passes=False),
    out_type=[
        jax.ShapeDtypeStruct((NW, 3, 16), jnp.float32),
        jax.ShapeDtypeStruct((NW, 3, 16), jnp.int32),
    ],
    scratch_types=[
        pltpu.VMEM((D,), jnp.float32),
        pltpu.VMEM((CHUNK * D,), jnp.float32),
        pltpu.VMEM((CHUNK * D,), jnp.float32),
        pltpu.VMEM((3, 16), jnp.float32),
        pltpu.VMEM((3, 16), jnp.int32),
        pltpu.SemaphoreType.DMA,
        pltpu.SemaphoreType.DMA,
    ],
)
def _phase1(
    x_hbm, w_hbm, vals_hbm, idx_hbm, x_v, buf0, buf1, bv_v, bi_v, sem0, sem1
):
    bufs = (buf0, buf1)
    sems = (sem0, sem1)
    wid = lax.axis_index("s") * NC + lax.axis_index("c")
    row0 = wid * ROWS_PER_W

    pltpu.sync_copy(x_hbm, x_v)

    lane = lax.iota(jnp.int32, 16)
    zero = jnp.zeros((16,), jnp.float32)

    def dma(c, b):
        return pltpu.make_async_copy(
            w_hbm.at[pl.ds((row0 + c * CHUNK) * D, CHUNK * D)],
            bufs[b],
            sems[b],
        )

    dma(0, 0).start()
    dma(1, 1).start()

    def process_chunk(b, c, carry):
        def sub_body(s16, carry):
            bv0, bv1, bv2, bi0, bi1, bi2 = carry
            fbase = (s16 * 16 + lane) * D

            def dc_body(dc, accs):
                a = list(accs)
                idx0 = fbase + jnp.broadcast_to(dc * 16, (16,))
                xc = x_v[pl.ds(dc * 16, 16)]
                for dd in range(16):
                    g = plsc.load_gather(
                        bufs[b], [idx0 + jnp.full((16,), dd, jnp.int32)]
                    )
                    xb = jnp.broadcast_to(xc[dd], (16,))
                    u = dd % 4
                    a[u] = a[u] + g * g
                    a[4 + u] = a[4 + u] + g * xb
                return tuple(a)

            accs = lax.fori_loop(0, DCH, dc_body, (zero,) * 8)
            sq = (accs[0] + accs[1]) + (accs[2] + accs[3])
            dt = (accs[4] + accs[5]) + (accs[6] + accs[7])
            d2 = sq - 2.0 * dt
            giv = jnp.broadcast_to(
                (row0 + c * CHUNK + s16 * 16).astype(jnp.int32), (16,)
            ) + lane
            lt0 = d2 < bv0
            lt1 = d2 < bv1
            lt2 = d2 < bv2
            nb0 = jnp.where(lt0, d2, bv0)
            ni0 = jnp.where(lt0, giv, bi0)
            nb1 = jnp.where(lt0, bv0, jnp.where(lt1, d2, bv1))
            ni1 = jnp.where(lt0, bi0, jnp.where(lt1, giv, bi1))
            nb2 = jnp.where(lt1, bv1, jnp.where(lt2, d2, bv2))
            ni2 = jnp.where(lt1, bi1, jnp.where(lt2, giv, bi2))
            return nb0, nb1, nb2, ni0, ni1, ni2

        return lax.fori_loop(0, NSUB, sub_body, carry)

    def group_body(g, carry):
        for b in range(NBUF):
            c = g * NBUF + b
            dma(c, b).wait()
            carry = process_chunk(b, c, carry)

            @pl.when(g < NGROUPS - 1)
            def _():
                dma(c + NBUF, b).start()

        return carry

    infv = jnp.full((16,), jnp.inf, jnp.float32)
    zidx = jnp.zeros((16,), jnp.int32)
    carry = lax.fori_loop(
        0, NGROUPS, group_body, (infv, infv, infv, zidx, zidx, zidx)
    )
    bv0, bv1, bv2, bi0, bi1, bi2 = carry

    bv_v[0, :] = bv0
    bv_v[1, :] = bv1
    bv_v[2, :] = bv2
    bi_v[0, :] = bi0
    bi_v[1, :] = bi1
    bi_v[2, :] = bi2
    pltpu.sync_copy(bv_v, vals_hbm.at[wid])
    pltpu.sync_copy(bi_v, idx_hbm.at[wid])


def _merge_body(vals_ref, idx_ref, probs_ref, x_ref, k_ref, out_ref):
    v = vals_ref[...].reshape(NW * 3, 16)
    ii = idx_ref[...].reshape(NW * 3, 16)
    lin = (
        lax.broadcasted_iota(jnp.int32, (NW * 3, 16), 0) * 16
        + lax.broadcasted_iota(jnp.int32, (NW * 3, 16), 1)
    )
    probs = probs_ref[...]
    fiota = (
        lax.broadcasted_iota(jnp.int32, probs.shape, 0) * probs.shape[1]
        + lax.broadcasted_iota(jnp.int32, probs.shape, 1)
    )
    xx = x_ref[...]
    xsq = jnp.sum(xx * xx)
    kv = k_ref[0, 0]

    num = jnp.float32(0.0)
    den = jnp.float32(0.0)
    for t in range(3):
        m = jnp.min(v)
        pos = jnp.min(jnp.where(v == m, lin, jnp.int32(2**30)))
        sel = lin == pos
        gidx = jnp.sum(jnp.where(sel, ii, 0))
        d2 = jnp.maximum(m + xsq, 0.0)
        e = jnp.exp(jnp.sqrt(d2))
        p = jnp.sum(jnp.where(fiota == gidx, probs, 0.0))
        num = num + jnp.where(t < kv, p * e, 0.0)
        den = den + e
        v = jnp.where(sel, jnp.float32(jnp.inf), v)

    out_ref[...] = jnp.broadcast_to(num / den, (1, 1))


def _phase2(vals, idxs, probs2d, x2d, k_arr):
    return pl.pallas_call(
        _merge_body,
        out_shape=jax.ShapeDtypeStruct((1, 1), jnp.float32),
    )(vals, idxs, probs2d, x2d, k_arr)


def kernel(x, weights, probabilities, k):
    w_flat = weights.reshape(SIZE * D)
    vals, idxs = _phase1(x, w_flat)
    probs2d = probabilities.reshape(D, SIZE // D)
    x2d = x.reshape(1, D)
    k_arr = jnp.asarray(k, jnp.int32).reshape(1, 1)
    out = _phase2(vals, idxs, probs2d, x2d, k_arr)
    return out[0, 0]


# speedup vs baseline: 4.8430x; 4.8430x over previous
import functools

import jax
import jax.numpy as jnp
from jax import lax
from jax.experimental import pallas as pl
from jax.experimental.pallas import tpu as pltpu
from jax.experimental.pallas import tpu_sc as plsc

D = 512
SIZE = 65536
NC = 2
NS = 16
NW = NC * NS
ROWS_PER_W = SIZE // NW
CHUNK = 64
NBUF = 2
NCHUNK = ROWS_PER_W // CHUNK
NGROUPS = NCHUNK // NBUF
NSUB = CHUNK // 16
DCH = D // 16
SKEW = 17

_mesh = plsc.VectorSubcoreMesh(core_axis_name="c", subcore_axis_name="s")


@functools.partial(
    pl.kernel,
    mesh=_mesh,
    compiler_params=pltpu.CompilerParams(needs_layout_passes=False),
    out_type=[
        jax.ShapeDtypeStruct((NW, 3, 16), jnp.float32),
        jax.ShapeDtypeStruct((NW, 3, 16), jnp.int32),
    ],
    scratch_types=[
        pltpu.VMEM((D,), jnp.float32),
        pltpu.VMEM((CHUNK, D), jnp.float32),
        pltpu.VMEM((CHUNK, D), jnp.float32),
        pltpu.VMEM((3, 16), jnp.float32),
        pltpu.VMEM((3, 16), jnp.int32),
        pltpu.SemaphoreType.DMA,
        pltpu.SemaphoreType.DMA,
    ],
)
def _phase1(
    x_hbm, w_hbm, vals_hbm, idx_hbm,
    x_v, buf0, buf1, bv_v, bi_v, sem0, sem1,
):
    bufs = (buf0, buf1)
    sems = (sem0, sem1)
    wid = lax.axis_index("s") * NC + lax.axis_index("c")
    row0 = wid * ROWS_PER_W

    pltpu.sync_copy(x_hbm, x_v)

    lane = lax.iota(jnp.int32, 16)
    zero = jnp.zeros((16,), jnp.float32)
    k17 = lane * SKEW
    rowvs = [lane + 16 * sg for sg in range(NSUB)]

    def dma(c, b):
        return pltpu.make_async_copy(
            w_hbm.at[pl.ds(row0 + c * CHUNK, CHUNK), :],
            bufs[b],
            sems[b],
        )

    dma(0, 0).start()
    dma(1, 1).start()

    def process_chunk(b, c, carry):
        def dc_body(dc, accs):
            a = list(accs)
            for dd in range(16):
                colv = (jnp.broadcast_to(dc * 16 + dd, (16,)) + k17) & 511
                xv = plsc.load_gather(x_v, [colv])
                for sg in range(NSUB):
                    g = plsc.load_gather(bufs[b], [rowvs[sg], colv])
                    dlt = g - xv
                    a[2 * sg + (dd & 1)] = a[2 * sg + (dd & 1)] + dlt * dlt
            return tuple(a)

        accs = lax.fori_loop(0, DCH, dc_body, (zero,) * (2 * NSUB))

        bv0, bv1, bv2, bi0, bi1, bi2 = carry
        for sg in range(NSUB):
            d2 = accs[2 * sg] + accs[2 * sg + 1]
            giv = jnp.broadcast_to(
                (row0 + c * CHUNK + sg * 16).astype(jnp.int32), (16,)
            ) + lane
            lt0 = d2 < bv0
            lt1 = d2 < bv1
            lt2 = d2 < bv2
            nb0 = jnp.where(lt0, d2, bv0)
            ni0 = jnp.where(lt0, giv, bi0)
            nb1 = jnp.where(lt0, bv0, jnp.where(lt1, d2, bv1))
            ni1 = jnp.where(lt0, bi0, jnp.where(lt1, giv, bi1))
            nb2 = jnp.where(lt1, bv1, jnp.where(lt2, d2, bv2))
            ni2 = jnp.where(lt1, bi1, jnp.where(lt2, giv, bi2))
            bv0, bv1, bv2, bi0, bi1, bi2 = nb0, nb1, nb2, ni0, ni1, ni2
        return bv0, bv1, bv2, bi0, bi1, bi2

    def group_body(g, carry):
        for b in range(NBUF):
            c = g * NBUF + b
            dma(c, b).wait()
            carry = process_chunk(b, c, carry)

            @pl.when(g < NGROUPS - 1)
            def _():
                dma(c + NBUF, b).start()

        return carry

    infv = jnp.full((16,), jnp.inf, jnp.float32)
    zidx = jnp.zeros((16,), jnp.int32)
    carry = lax.fori_loop(
        0, NGROUPS, group_body, (infv, infv, infv, zidx, zidx, zidx)
    )
    bv0, bv1, bv2, bi0, bi1, bi2 = carry

    bv_v[0, :] = bv0
    bv_v[1, :] = bv1
    bv_v[2, :] = bv2
    bi_v[0, :] = bi0
    bi_v[1, :] = bi1
    bi_v[2, :] = bi2
    pltpu.sync_copy(bv_v, vals_hbm.at[wid])
    pltpu.sync_copy(bi_v, idx_hbm.at[wid])


def _merge_body(vals_ref, idx_ref, probs_ref, k_ref, out_ref):
    v = vals_ref[...].reshape(NW * 3, 16)
    ii = idx_ref[...].reshape(NW * 3, 16)
    lin = (
        lax.broadcasted_iota(jnp.int32, (NW * 3, 16), 0) * 16
        + lax.broadcasted_iota(jnp.int32, (NW * 3, 16), 1)
    )
    probs = probs_ref[...]
    fiota = (
        lax.broadcasted_iota(jnp.int32, probs.shape, 0) * probs.shape[1]
        + lax.broadcasted_iota(jnp.int32, probs.shape, 1)
    )
    kv = k_ref[0, 0]

    num = jnp.float32(0.0)
    den = jnp.float32(0.0)
    for t in range(3):
        m = jnp.min(v)
        pos = jnp.min(jnp.where(v == m, lin, jnp.int32(2**30)))
        sel = lin == pos
        gidx = jnp.sum(jnp.where(sel, ii, 0))
        e = jnp.exp(jnp.sqrt(jnp.maximum(m, 0.0)))
        p = jnp.sum(jnp.where(fiota == gidx, probs, 0.0))
        num = num + jnp.where(t < kv, p * e, 0.0)
        den = den + e
        v = jnp.where(sel, jnp.float32(jnp.inf), v)

    out_ref[...] = jnp.broadcast_to(num / den, (1, 1))


def _phase2(vals, idxs, probs2d, k_arr):
    return pl.pallas_call(
        _merge_body,
        out_shape=jax.ShapeDtypeStruct((1, 1), jnp.float32),
    )(vals, idxs, probs2d, k_arr)


def kernel(x, weights, probabilities, k):
    vals, idxs = _phase1(x, weights)
    probs2d = probabilities.reshape(D, SIZE // D)
    k_arr = jnp.asarray(k, jnp.int32).reshape(1, 1)
    out = _phase2(vals, idxs, probs2d, k_arr)
    return out[0, 0]


# speedup vs baseline: 8.4973x; 1.7546x over previous
import functools

import jax
import jax.numpy as jnp
from jax import lax
from jax.experimental import pallas as pl
from jax.experimental.pallas import tpu as pltpu
from jax.experimental.pallas import tpu_sc as plsc

D = 512
SIZE = 65536
SC_SIZE = 20480
TC_SIZE = SIZE - SC_SIZE
TC_BLK = 2048
NC = 2
NS = 16
NW = NC * NS
ROWS_PER_W = SC_SIZE // NW
CHUNK = 64
NBUF = 2
NCHUNK = ROWS_PER_W // CHUNK
NGROUPS = NCHUNK // NBUF
NSUB = CHUNK // 16
DCH = D // 16
SKEW = 17

_mesh = plsc.VectorSubcoreMesh(core_axis_name="c", subcore_axis_name="s")


@functools.partial(
    pl.kernel,
    mesh=_mesh,
    compiler_params=pltpu.CompilerParams(needs_layout_passes=False),
    out_type=[
        jax.ShapeDtypeStruct((NW, 3, 16), jnp.float32),
        jax.ShapeDtypeStruct((NW, 3, 16), jnp.int32),
    ],
    scratch_types=[
        pltpu.VMEM((D,), jnp.float32),
        pltpu.VMEM((CHUNK, D), jnp.float32),
        pltpu.VMEM((CHUNK, D), jnp.float32),
        pltpu.VMEM((3, 16), jnp.float32),
        pltpu.VMEM((3, 16), jnp.int32),
        pltpu.SemaphoreType.DMA,
        pltpu.SemaphoreType.DMA,
    ],
)
def _phase1(
    x_hbm, w_hbm, vals_hbm, idx_hbm,
    x_v, buf0, buf1, bv_v, bi_v, sem0, sem1,
):
    bufs = (buf0, buf1)
    sems = (sem0, sem1)
    wid = lax.axis_index("s") * NC + lax.axis_index("c")
    row0 = wid * ROWS_PER_W

    pltpu.sync_copy(x_hbm, x_v)

    lane = lax.iota(jnp.int32, 16)
    zero = jnp.zeros((16,), jnp.float32)
    k17 = lane * SKEW
    rowvs = [lane + 16 * sg for sg in range(NSUB)]

    H = CHUNK // 2

    def dma_halves(c, b):
        r = row0 + c * CHUNK
        return (
            pltpu.make_async_copy(
                w_hbm.at[pl.ds(r, H), :], bufs[b].at[pl.ds(0, H), :], sems[b]
            ),
            pltpu.make_async_copy(
                w_hbm.at[pl.ds(r + H, H), :],
                bufs[b].at[pl.ds(H, H), :],
                sems[b],
            ),
        )

    def dma_start(c, b):
        for h in dma_halves(c, b):
            h.start()

    def dma_wait(c, b):
        for h in dma_halves(c, b):
            h.wait()

    dma_start(0, 0)
    dma_start(1, 1)

    def process_chunk(b, c, carry):
        def dc_body(dc, accs):
            a = list(accs)
            for dd in range(8):
                colv = (jnp.broadcast_to(dc * 8 + dd, (16,)) + k17) & 511
                xv = plsc.load_gather(x_v, [colv])
                for sg in range(NSUB):
                    g = plsc.load_gather(bufs[b], [rowvs[sg], colv])
                    dlt = g - xv
                    a[2 * sg + (dd & 1)] = a[2 * sg + (dd & 1)] + dlt * dlt
            return tuple(a)

        accs = lax.fori_loop(0, D // 8, dc_body, (zero,) * (2 * NSUB))

        bv0, bv1, bv2, bi0, bi1, bi2 = carry
        for sg in range(NSUB):
            d2 = accs[2 * sg] + accs[2 * sg + 1]
            giv = jnp.broadcast_to(
                (row0 + c * CHUNK + sg * 16).astype(jnp.int32), (16,)
            ) + lane
            lt0 = d2 < bv0
            lt1 = d2 < bv1
            lt2 = d2 < bv2
            nb0 = jnp.where(lt0, d2, bv0)
            ni0 = jnp.where(lt0, giv, bi0)
            nb1 = jnp.where(lt0, bv0, jnp.where(lt1, d2, bv1))
            ni1 = jnp.where(lt0, bi0, jnp.where(lt1, giv, bi1))
            nb2 = jnp.where(lt1, bv1, jnp.where(lt2, d2, bv2))
            ni2 = jnp.where(lt1, bi1, jnp.where(lt2, giv, bi2))
            bv0, bv1, bv2, bi0, bi1, bi2 = nb0, nb1, nb2, ni0, ni1, ni2
        return bv0, bv1, bv2, bi0, bi1, bi2

    def group_body(g, carry):
        for b in range(NBUF):
            c = g * NBUF + b
            dma_wait(c, b)
            carry = process_chunk(b, c, carry)

            @pl.when(g < NGROUPS - 1)
            def _():
                dma_start(c + NBUF, b)

        return carry

    infv = jnp.full((16,), jnp.inf, jnp.float32)
    zidx = jnp.zeros((16,), jnp.int32)
    carry = lax.fori_loop(
        0, NGROUPS, group_body, (infv, infv, infv, zidx, zidx, zidx)
    )
    bv0, bv1, bv2, bi0, bi1, bi2 = carry

    bv_v[0, :] = bv0
    bv_v[1, :] = bv1
    bv_v[2, :] = bv2
    bi_v[0, :] = bi0
    bi_v[1, :] = bi1
    bi_v[2, :] = bi2
    pltpu.sync_copy(bv_v, vals_hbm.at[wid])
    pltpu.sync_copy(bi_v, idx_hbm.at[wid])


def _tc_dist_body(x_ref, w_ref, out_ref):
    w = w_ref[...]
    xx = x_ref[...]
    dlt = w - xx
    out_ref[...] = jnp.sum(dlt * dlt, axis=1, keepdims=True).reshape(1, 1, TC_BLK)


def _tc_dist(x2d, weights):
    nblk = TC_SIZE // TC_BLK
    return pl.pallas_call(
        _tc_dist_body,
        grid=(nblk,),
        in_specs=[
            pl.BlockSpec((1, D), lambda i: (0, 0)),
            pl.BlockSpec((TC_BLK, D), lambda i: (i + SC_SIZE // TC_BLK, 0)),
        ],
        out_specs=pl.BlockSpec((1, 1, TC_BLK), lambda i: (i, 0, 0)),
        out_shape=jax.ShapeDtypeStruct((nblk, 1, TC_BLK), jnp.float32),
    )(x2d, weights)


def _merge_body(vals_ref, idx_ref, d2tc_ref, probs_ref, k_ref, out_ref):
    v = vals_ref[...].reshape(NW * 3, 16)
    ii = idx_ref[...].reshape(NW * 3, 16)
    lin = (
        lax.broadcasted_iota(jnp.int32, (NW * 3, 16), 0) * 16
        + lax.broadcasted_iota(jnp.int32, (NW * 3, 16), 1)
    )
    vt = d2tc_ref[...].reshape(TC_SIZE // TC_BLK, TC_BLK)
    lint = (
        lax.broadcasted_iota(jnp.int32, vt.shape, 0) * vt.shape[1]
        + lax.broadcasted_iota(jnp.int32, vt.shape, 1)
    )
    probs = probs_ref[...]
    fiota = (
        lax.broadcasted_iota(jnp.int32, probs.shape, 0) * probs.shape[1]
        + lax.broadcasted_iota(jnp.int32, probs.shape, 1)
    )
    kv = k_ref[0, 0]

    big = jnp.int32(2**30)
    num = jnp.float32(0.0)
    den = jnp.float32(0.0)
    for t in range(3):
        ms = jnp.min(v)
        mt = jnp.min(vt)
        take_sc = ms <= mt
        m = jnp.where(take_sc, ms, mt)
        pos_s = jnp.min(jnp.where(v == ms, lin, big))
        g_s = jnp.sum(jnp.where(lin == pos_s, ii, 0))
        pos_t = jnp.min(jnp.where(vt == mt, lint, big))
        gidx = jnp.where(take_sc, g_s, SC_SIZE + pos_t)
        e = jnp.exp(jnp.sqrt(jnp.maximum(m, 0.0)))
        p = jnp.sum(jnp.where(fiota == gidx, probs, 0.0))
        num = num + jnp.where(t < kv, p * e, 0.0)
        den = den + e
        v = jnp.where((lin == pos_s) & take_sc, jnp.float32(jnp.inf), v)
        vt = jnp.where(
            (lint == pos_t) & jnp.logical_not(take_sc), jnp.float32(jnp.inf), vt
        )

    out_ref[...] = jnp.broadcast_to(num / den, (1, 1))


def _phase2(vals, idxs, d2tc, probs2d, k_arr):
    return pl.pallas_call(
        _merge_body,
        out_shape=jax.ShapeDtypeStruct((1, 1), jnp.float32),
    )(vals, idxs, d2tc, probs2d, k_arr)


def kernel(x, weights, probabilities, k):
    x2d = x.reshape(1, D)
    vals, idxs = _phase1(x, weights)
    d2tc = _tc_dist(x2d, weights)
    probs2d = probabilities.reshape(D, SIZE // D)
    k_arr = jnp.asarray(k, jnp.int32).reshape(1, 1)
    out = _phase2(vals, idxs, d2tc, probs2d, k_arr)
    return out[0, 0]


# speedup vs baseline: 8.6718x; 1.0205x over previous
import functools

import jax
import jax.numpy as jnp
from jax import lax
from jax.experimental import pallas as pl
from jax.experimental.pallas import tpu as pltpu
from jax.experimental.pallas import tpu_sc as plsc

D = 512
SIZE = 65536
SC_SIZE = 24576
TC_SIZE = SIZE - SC_SIZE
TC_BLK = 2048
NC = 2
NS = 16
NW = NC * NS
ROWS_PER_W = SC_SIZE // NW
CHUNK = 64
NBUF = 2
NCHUNK = ROWS_PER_W // CHUNK
NGROUPS = NCHUNK // NBUF
NSUB = CHUNK // 16
DCH = D // 16
SKEW = 17

_mesh = plsc.VectorSubcoreMesh(core_axis_name="c", subcore_axis_name="s")


@functools.partial(
    pl.kernel,
    mesh=_mesh,
    compiler_params=pltpu.CompilerParams(needs_layout_passes=False),
    out_type=[
        jax.ShapeDtypeStruct((NW, 3, 16), jnp.float32),
        jax.ShapeDtypeStruct((NW, 3, 16), jnp.int32),
    ],
    scratch_types=[
        pltpu.VMEM((D,), jnp.float32),
        pltpu.VMEM((CHUNK, D), jnp.float32),
        pltpu.VMEM((CHUNK, D), jnp.float32),
        pltpu.VMEM((3, 16), jnp.float32),
        pltpu.VMEM((3, 16), jnp.int32),
        pltpu.SemaphoreType.DMA,
        pltpu.SemaphoreType.DMA,
    ],
)
def _phase1(
    x_hbm, w_hbm, vals_hbm, idx_hbm,
    x_v, buf0, buf1, bv_v, bi_v, sem0, sem1,
):
    bufs = (buf0, buf1)
    sems = (sem0, sem1)
    wid = lax.axis_index("s") * NC + lax.axis_index("c")
    row0 = wid * ROWS_PER_W

    pltpu.sync_copy(x_hbm, x_v)

    lane = lax.iota(jnp.int32, 16)
    zero = jnp.zeros((16,), jnp.float32)
    k17 = lane * SKEW
    rowvs = [lane + 16 * sg for sg in range(NSUB)]

    H = CHUNK // 2

    def dma_halves(c, b):
        r = row0 + c * CHUNK
        return (
            pltpu.make_async_copy(
                w_hbm.at[pl.ds(r, H), :], bufs[b].at[pl.ds(0, H), :], sems[b]
            ),
            pltpu.make_async_copy(
                w_hbm.at[pl.ds(r + H, H), :],
                bufs[b].at[pl.ds(H, H), :],
                sems[b],
            ),
        )

    def dma_start(c, b):
        for h in dma_halves(c, b):
            h.start()

    def dma_wait(c, b):
        for h in dma_halves(c, b):
            h.wait()

    dma_start(0, 0)
    dma_start(1, 1)

    def process_chunk(b, c, carry):
        def dc_body(dc, accs):
            a = list(accs)
            for dd in range(8):
                colv = (jnp.broadcast_to(dc * 8 + dd, (16,)) + k17) & 511
                xv = plsc.load_gather(x_v, [colv])
                for sg in range(NSUB):
                    g = plsc.load_gather(bufs[b], [rowvs[sg], colv])
                    dlt = g - xv
                    a[2 * sg + (dd & 1)] = a[2 * sg + (dd & 1)] + dlt * dlt
            return tuple(a)

        accs = lax.fori_loop(0, D // 8, dc_body, (zero,) * (2 * NSUB))

        bv0, bv1, bv2, bi0, bi1, bi2 = carry
        for sg in range(NSUB):
            d2 = accs[2 * sg] + accs[2 * sg + 1]
            giv = jnp.broadcast_to(
                (row0 + c * CHUNK + sg * 16).astype(jnp.int32), (16,)
            ) + lane
            lt0 = d2 < bv0
            lt1 = d2 < bv1
            lt2 = d2 < bv2
            nb0 = jnp.where(lt0, d2, bv0)
            ni0 = jnp.where(lt0, giv, bi0)
            nb1 = jnp.where(lt0, bv0, jnp.where(lt1, d2, bv1))
            ni1 = jnp.where(lt0, bi0, jnp.where(lt1, giv, bi1))
            nb2 = jnp.where(lt1, bv1, jnp.where(lt2, d2, bv2))
            ni2 = jnp.where(lt1, bi1, jnp.where(lt2, giv, bi2))
            bv0, bv1, bv2, bi0, bi1, bi2 = nb0, nb1, nb2, ni0, ni1, ni2
        return bv0, bv1, bv2, bi0, bi1, bi2

    def group_body(g, carry):
        for b in range(NBUF):
            c = g * NBUF + b
            dma_wait(c, b)
            carry = process_chunk(b, c, carry)

            @pl.when(g < NGROUPS - 1)
            def _():
                dma_start(c + NBUF, b)

        return carry

    infv = jnp.full((16,), jnp.inf, jnp.float32)
    zidx = jnp.zeros((16,), jnp.int32)
    carry = lax.fori_loop(
        0, NGROUPS, group_body, (infv, infv, infv, zidx, zidx, zidx)
    )
    bv0, bv1, bv2, bi0, bi1, bi2 = carry

    bv_v[0, :] = bv0
    bv_v[1, :] = bv1
    bv_v[2, :] = bv2
    bi_v[0, :] = bi0
    bi_v[1, :] = bi1
    bi_v[2, :] = bi2
    pltpu.sync_copy(bv_v, vals_hbm.at[wid])
    pltpu.sync_copy(bi_v, idx_hbm.at[wid])


def _tc_dist_body(x_ref, w_ref, out_ref):
    w = w_ref[...]
    xx = x_ref[...]
    dlt = w - xx
    out_ref[...] = jnp.sum(dlt * dlt, axis=1, keepdims=True).reshape(1, 1, TC_BLK)


def _tc_dist(x2d, weights):
    nblk = TC_SIZE // TC_BLK
    return pl.pallas_call(
        _tc_dist_body,
        grid=(nblk,),
        in_specs=[
            pl.BlockSpec((1, D), lambda i: (0, 0)),
            pl.BlockSpec((TC_BLK, D), lambda i: (i + SC_SIZE // TC_BLK, 0)),
        ],
        out_specs=pl.BlockSpec((1, 1, TC_BLK), lambda i: (i, 0, 0)),
        out_shape=jax.ShapeDtypeStruct((nblk, 1, TC_BLK), jnp.float32),
    )(x2d, weights)


def _merge_body(vals_ref, idx_ref, d2tc_ref, probs_ref, k_ref, out_ref):
    v = vals_ref[...].reshape(NW * 3, 16)
    ii = idx_ref[...].reshape(NW * 3, 16)
    lin = (
        lax.broadcasted_iota(jnp.int32, (NW * 3, 16), 0) * 16
        + lax.broadcasted_iota(jnp.int32, (NW * 3, 16), 1)
    )
    vt = d2tc_ref[...].reshape(TC_SIZE // TC_BLK, TC_BLK)
    lint = (
        lax.broadcasted_iota(jnp.int32, vt.shape, 0) * vt.shape[1]
        + lax.broadcasted_iota(jnp.int32, vt.shape, 1)
    )
    probs = probs_ref[...]
    fiota = (
        lax.broadcasted_iota(jnp.int32, probs.shape, 0) * probs.shape[1]
        + lax.broadcasted_iota(jnp.int32, probs.shape, 1)
    )
    kv = k_ref[0, 0]

    big = jnp.int32(2**30)
    num = jnp.float32(0.0)
    den = jnp.float32(0.0)
    for t in range(3):
        ms = jnp.min(v)
        mt = jnp.min(vt)
        take_sc = ms <= mt
        m = jnp.where(take_sc, ms, mt)
        pos_s = jnp.min(jnp.where(v == ms, lin, big))
        g_s = jnp.sum(jnp.where(lin == pos_s, ii, 0))
        pos_t = jnp.min(jnp.where(vt == mt, lint, big))
        gidx = jnp.where(take_sc, g_s, SC_SIZE + pos_t)
        e = jnp.exp(jnp.sqrt(jnp.maximum(m, 0.0)))
        p = jnp.sum(jnp.where(fiota == gidx, probs, 0.0))
        num = num + jnp.where(t < kv, p * e, 0.0)
        den = den + e
        v = jnp.where((lin == pos_s) & take_sc, jnp.float32(jnp.inf), v)
        vt = jnp.where(
            (lint == pos_t) & jnp.logical_not(take_sc), jnp.float32(jnp.inf), vt
        )

    out_ref[...] = jnp.broadcast_to(num / den, (1, 1))


def _phase2(vals, idxs, d2tc, probs2d, k_arr):
    return pl.pallas_call(
        _merge_body,
        out_shape=jax.ShapeDtypeStruct((1, 1), jnp.float32),
    )(vals, idxs, d2tc, probs2d, k_arr)


def kernel(x, weights, probabilities, k):
    x2d = x.reshape(1, D)
    vals, idxs = _phase1(x, weights)
    d2tc = _tc_dist(x2d, weights)
    probs2d = probabilities.reshape(D, SIZE // D)
    k_arr = jnp.asarray(k, jnp.int32).reshape(1, 1)
    out = _phase2(vals, idxs, d2tc, probs2d, k_arr)
    return out[0, 0]
